# baseline (speedup 1.0000x reference)
import numpy as np

import jax
import jax.numpy as jnp
from jax.experimental import pallas as pl
from jax.experimental.pallas import tpu as pltpu

_BN = 128


def _phase_filters(w_packed, cin):
    dr = np.arange(6)
    dc = np.arange(6)
    ci = np.arange(cin)
    idx = (((dr[:, None, None] // 2) * 3 + dc[None, :, None] // 2) * 4
           + 2 * (dr[:, None, None] % 2) + dc[None, :, None] % 2) * 8 + ci
    return jnp.transpose(w_packed[:, :, idx], (0, 1, 4, 2, 3))


def _conv1_toeplitz(w1):
    k1 = _phase_filters(w1, 3)
    r = np.arange(8)[None, :, None]
    a = np.arange(6)[None, None, :]
    par = np.arange(2)[:, None, None]
    rsel = (r == a + 2 * par).astype(np.float32)
    w = np.arange(32)[:, None, None]
    d = np.arange(6)[None, :, None]
    k = np.arange(16)[None, None, :]
    csel = ((w == 2 * k + d) & (k <= 13)).astype(np.float32)
    t = jnp.einsum('qocad,pra,wdk->pcrwqok', k1,
                   jnp.asarray(rsel), jnp.asarray(csel))
    t = t.reshape(2, 768, 4, 96)
    return jnp.pad(t, ((0, 0), (0, 0), (0, 0), (0, 32))).reshape(2, 768, 512)


def _conv2_toeplitz(w2):
    k2 = _phase_filters(w2, 6)
    w = np.arange(16)[:, None, None]
    d = np.arange(6)[None, :, None]
    k = np.arange(8)[None, None, :]
    csel = ((w == 2 * k + d) & (k <= 4)).astype(np.float32)
    t = jnp.einsum('qocad,wdk->acwqok', k2, jnp.asarray(csel))
    return jnp.pad(t.reshape(6, 96, 512), ((0, 0), (0, 32), (0, 0)))


def _fused_kernel(x_ref, t1_ref, b1_ref, t2_ref, b2_ref,
                  f1_ref, c1_ref, f2_ref, c2_ref, f3_ref, c3_ref, o_ref):
    X = x_ref[...]

    def chunk_max(z):
        return jnp.maximum(jnp.maximum(z[:, :128], z[:, 128:256]),
                           jnp.maximum(z[:, 256:384], z[:, 384:]))

    rows = []
    for h in range(14):
        base = 128 * (h // 2)
        win = jnp.concatenate(
            [X[:, c * 1024 + base:c * 1024 + base + 256] for c in range(3)],
            axis=1)
        z = jnp.dot(win, t1_ref[h % 2], preferred_element_type=jnp.float32)
        rows.append(chunk_max(z))
    p1 = jnp.stack(rows, axis=0)
    p1 = jnp.maximum(p1 + b1_ref[...], 0.0)

    rows2 = []
    for h in range(5):
        z = jnp.dot(p1[2 * h], t2_ref[0], preferred_element_type=jnp.float32)
        for a in range(1, 6):
            z = z + jnp.dot(p1[2 * h + a], t2_ref[a],
                            preferred_element_type=jnp.float32)
        rows2.append(chunk_max(z))
    p2 = jnp.stack(rows2, axis=0)
    p2 = jnp.maximum(p2 + b2_ref[...], 0.0)

    h1 = jnp.dot(p2[0], f1_ref[0], preferred_element_type=jnp.float32)
    for h in range(1, 5):
        h1 = h1 + jnp.dot(p2[h], f1_ref[h], preferred_element_type=jnp.float32)
    h1 = jnp.maximum(h1 + c1_ref[...], 0.0)
    h2 = jnp.dot(h1, f2_ref[...], preferred_element_type=jnp.float32)
    h2 = jnp.maximum(h2 + c2_ref[...], 0.0)
    z3 = jnp.dot(h2, f3_ref[...], preferred_element_type=jnp.float32)
    z3 = z3 + c3_ref[...]
    m = jnp.max(z3, axis=1, keepdims=True)
    e = jnp.exp(z3 - m)
    s = jnp.sum(e, axis=1, keepdims=True)
    o_ref[...] = z3 - m - jnp.log(s)


def kernel(x, w1, b1, w2, b2, wf1, bf1, wf2, bf2, wf3, bf3):
    n = x.shape[0]
    n_pad = ((n + _BN - 1) // _BN) * _BN
    if n_pad != n:
        x = jnp.pad(x, ((0, n_pad - n), (0, 0), (0, 0), (0, 0)))
    x2d = x.reshape(n_pad, 3072)

    t1 = _conv1_toeplitz(w1)
    t2 = _conv2_toeplitz(w2)
    b1l = jnp.pad((jnp.tile(b1, (1, 16)) *
                   (np.arange(16) < 14)).reshape(1, 96), ((0, 0), (0, 32)))
    b2l = (jnp.tile(b2, (1, 8)) * (np.arange(8) < 5)).reshape(1, 128)
    f1 = jnp.pad(wf1.reshape(16, 5, 5, 120),
                 ((0, 0), (0, 0), (0, 3), (0, 0)))
    f1 = jnp.transpose(f1, (1, 0, 2, 3)).reshape(5, 128, 120)

    def whole(shape):
        nd = len(shape)
        return pl.BlockSpec(shape, lambda i, nd=nd: (0,) * nd)

    out = pl.pallas_call(
        _fused_kernel,
        out_shape=jax.ShapeDtypeStruct((n_pad, 10), jnp.float32),
        grid=(n_pad // _BN,),
        in_specs=[pl.BlockSpec((_BN, 3072), lambda i: (i, 0)),
                  whole(t1.shape), whole(b1l.shape),
                  whole(t2.shape), whole(b2l.shape),
                  whole(f1.shape), whole(bf1.shape),
                  whole(wf2.shape), whole(bf2.shape),
                  whole(wf3.shape), whole(bf3.shape)],
        out_specs=pl.BlockSpec((_BN, 10), lambda i: (i, 0)),
        compiler_params=pltpu.CompilerParams(
            dimension_semantics=("parallel",)),
    )(x2d, t1, b1l, t2, b2l, f1, bf1, wf2, bf2, wf3, bf3)
    return out[:n]


# speedup vs baseline: 1.0618x; 1.0618x over previous
import numpy as np

import jax
import jax.numpy as jnp
from jax.experimental import pallas as pl
from jax.experimental.pallas import tpu as pltpu

_BN = 256


def _phase_filters(w_packed, cin):
    dr = np.arange(6)
    dc = np.arange(6)
    ci = np.arange(cin)
    idx = (((dr[:, None, None] // 2) * 3 + dc[None, :, None] // 2) * 4
           + 2 * (dr[:, None, None] % 2) + dc[None, :, None] % 2) * 8 + ci
    return jnp.transpose(w_packed[:, :, idx], (0, 1, 4, 2, 3))


def _conv1_toeplitz(w1):
    k1 = _phase_filters(w1, 3)
    r = np.arange(8)[None, :, None]
    a = np.arange(6)[None, None, :]
    par = np.arange(2)[:, None, None]
    rsel = (r == a + 2 * par).astype(np.float32)
    w = np.arange(32)[:, None, None]
    d = np.arange(6)[None, :, None]
    k = np.arange(16)[None, None, :]
    csel = ((w == 2 * k + d) & (k <= 13)).astype(np.float32)
    t = jnp.einsum('qocad,pra,wdk->pcrwqok', k1,
                   jnp.asarray(rsel), jnp.asarray(csel))
    t = t.reshape(2, 768, 4, 96)
    return jnp.pad(t, ((0, 0), (0, 0), (0, 0), (0, 32))).reshape(2, 768, 512)


def _conv2_toeplitz(w2):
    k2 = _phase_filters(w2, 6)
    w = np.arange(16)[:, None, None]
    d = np.arange(6)[None, :, None]
    k = np.arange(8)[None, None, :]
    csel = ((w == 2 * k + d) & (k <= 4)).astype(np.float32)
    t = jnp.einsum('qocad,wdk->acwqok', k2, jnp.asarray(csel))
    return jnp.pad(t.reshape(6, 96, 512), ((0, 0), (0, 32), (0, 0)))


def _fused_kernel(x_ref, t1_ref, b1_ref, t2_ref, b2_ref,
                  f1_ref, c1_ref, f2_ref, c2_ref, f3_ref, c3_ref, o_ref):
    X = x_ref[...]

    def chunk_max(z):
        return jnp.maximum(jnp.maximum(z[:, :128], z[:, 128:256]),
                           jnp.maximum(z[:, 256:384], z[:, 384:]))

    rows = []
    for h in range(14):
        base = 128 * (h // 2)
        win = jnp.concatenate(
            [X[:, c * 1024 + base:c * 1024 + base + 256] for c in range(3)],
            axis=1)
        z = jnp.dot(win, t1_ref[h % 2], preferred_element_type=jnp.float32)
        rows.append(chunk_max(z))
    p1 = jnp.stack(rows, axis=0)
    p1 = jnp.maximum(p1 + b1_ref[...], 0.0)

    rows2 = []
    for h in range(5):
        z = jnp.dot(p1[2 * h], t2_ref[0], preferred_element_type=jnp.float32)
        for a in range(1, 6):
            z = z + jnp.dot(p1[2 * h + a], t2_ref[a],
                            preferred_element_type=jnp.float32)
        rows2.append(chunk_max(z))
    p2 = jnp.stack(rows2, axis=0)
    p2 = jnp.maximum(p2 + b2_ref[...], 0.0)

    h1 = jnp.dot(p2[0], f1_ref[0], preferred_element_type=jnp.float32)
    for h in range(1, 5):
        h1 = h1 + jnp.dot(p2[h], f1_ref[h], preferred_element_type=jnp.float32)
    h1 = jnp.maximum(h1 + c1_ref[...], 0.0)
    h2 = jnp.dot(h1, f2_ref[...], preferred_element_type=jnp.float32)
    h2 = jnp.maximum(h2 + c2_ref[...], 0.0)
    z3 = jnp.dot(h2, f3_ref[...], preferred_element_type=jnp.float32)
    z3 = z3 + c3_ref[...]
    m = jnp.max(z3, axis=1, keepdims=True)
    e = jnp.exp(z3 - m)
    s = jnp.sum(e, axis=1, keepdims=True)
    o_ref[...] = z3 - m - jnp.log(s)


def kernel(x, w1, b1, w2, b2, wf1, bf1, wf2, bf2, wf3, bf3):
    n = x.shape[0]
    n_pad = ((n + _BN - 1) // _BN) * _BN
    if n_pad != n:
        x = jnp.pad(x, ((0, n_pad - n), (0, 0), (0, 0), (0, 0)))
    x2d = x.reshape(n_pad, 3072)

    t1 = _conv1_toeplitz(w1)
    t2 = _conv2_toeplitz(w2)
    b1l = jnp.pad((jnp.tile(b1, (1, 16)) *
                   (np.arange(16) < 14)).reshape(1, 96), ((0, 0), (0, 32)))
    b2l = (jnp.tile(b2, (1, 8)) * (np.arange(8) < 5)).reshape(1, 128)
    f1 = jnp.pad(wf1.reshape(16, 5, 5, 120),
                 ((0, 0), (0, 0), (0, 3), (0, 0)))
    f1 = jnp.transpose(f1, (1, 0, 2, 3)).reshape(5, 128, 120)

    def whole(shape):
        nd = len(shape)
        return pl.BlockSpec(shape, lambda i, nd=nd: (0,) * nd)

    out = pl.pallas_call(
        _fused_kernel,
        out_shape=jax.ShapeDtypeStruct((n_pad, 10), jnp.float32),
        grid=(n_pad // _BN,),
        in_specs=[pl.BlockSpec((_BN, 3072), lambda i: (i, 0)),
                  whole(t1.shape), whole(b1l.shape),
                  whole(t2.shape), whole(b2l.shape),
                  whole(f1.shape), whole(bf1.shape),
                  whole(wf2.shape), whole(bf2.shape),
                  whole(wf3.shape), whole(bf3.shape)],
        out_specs=pl.BlockSpec((_BN, 10), lambda i: (i, 0)),
        compiler_params=pltpu.CompilerParams(
            dimension_semantics=("parallel",)),
    )(x2d, t1, b1l, t2, b2l, f1, bf1, wf2, bf2, wf3, bf3)
    return out[:n]


# speedup vs baseline: 1.1484x; 1.0815x over previous
import numpy as np

import jax
import jax.numpy as jnp
from jax.experimental import pallas as pl
from jax.experimental.pallas import tpu as pltpu

_BN = 128


def _tap_select(cin):
    sel = np.zeros((288, cin, 6, 6), np.float32)
    for a in range(6):
        for d in range(6):
            base = ((a // 2) * 3 + (d // 2)) * 4 + 2 * (a % 2) + d % 2
            for c in range(cin):
                sel[base * 8 + c, c, a, d] = 1.0
    return sel


def _col_select(win_w, kpad, k_max):
    w = np.arange(win_w)[:, None, None]
    d = np.arange(6)[None, :, None]
    k = np.arange(kpad)[None, None, :]
    return ((w == 2 * k + d) & (k <= k_max)).astype(np.float32)


def _toeplitz(w_packed, cin, win_w, kpad, k_max):
    cout = w_packed.shape[1]
    t = jnp.einsum('qot,tcad,wdk->qokacw', w_packed,
                   _tap_select(cin), _col_select(win_w, kpad, k_max))
    return t.reshape(4 * cout * kpad, 6 * cin * win_w)


def _fused_kernel(x_ref, t1_ref, b1_ref, t2_ref, b2_ref,
                  f1_ref, c1_ref, f2_ref, c2_ref, f3_ref, c3_ref, o_ref):
    bn = o_ref.shape[1]
    X = x_ref[...].reshape(32, 96, bn)

    t1 = t1_ref[...]
    rows = []
    for h in range(14):
        win = X[2 * h:2 * h + 6].reshape(576, bn)
        z = jnp.dot(t1, win, preferred_element_type=jnp.float32)
        rows.append(jnp.max(z.reshape(4, 96, bn), axis=0))
    y = jnp.stack(rows, axis=0)
    p1 = jnp.maximum(y + b1_ref[...], 0.0)

    t2 = t2_ref[...]
    rows2 = []
    for h in range(5):
        win = p1[2 * h:2 * h + 6].reshape(576, bn)
        z = jnp.dot(t2, win, preferred_element_type=jnp.float32)
        rows2.append(jnp.max(z.reshape(4, 128, bn), axis=0))
    y2 = jnp.stack(rows2, axis=0)
    p2 = jnp.maximum(y2 + b2_ref[...], 0.0)
    flat = p2.reshape(640, bn)

    h1 = jnp.dot(f1_ref[...], flat, preferred_element_type=jnp.float32)
    h1 = jnp.maximum(h1 + c1_ref[...], 0.0)
    h2 = jnp.dot(f2_ref[...], h1, preferred_element_type=jnp.float32)
    h2 = jnp.maximum(h2 + c2_ref[...], 0.0)
    z3 = jnp.dot(f3_ref[...], h2, preferred_element_type=jnp.float32)
    z3 = z3 + c3_ref[...]
    m = jnp.max(z3, axis=0, keepdims=True)
    e = jnp.exp(z3 - m)
    s = jnp.sum(e, axis=0, keepdims=True)
    o_ref[...] = z3 - m - jnp.log(s)


def kernel(x, w1, b1, w2, b2, wf1, bf1, wf2, bf2, wf3, bf3):
    n = x.shape[0]
    n_pad = ((n + _BN - 1) // _BN) * _BN
    if n_pad != n:
        x = jnp.pad(x, ((0, n_pad - n), (0, 0), (0, 0), (0, 0)))

    xt = jnp.transpose(x, (2, 1, 3, 0))

    t1 = _toeplitz(w1, 3, 32, 16, 13)
    t2 = _toeplitz(w2, 6, 16, 8, 4)
    b1s = (jnp.tile(b1, (1, 16)) *
           (np.arange(16) < 14).astype(np.float32)).reshape(96, 1)
    b2s = (jnp.tile(b2, (1, 8)) *
           (np.arange(8) < 5).astype(np.float32)).reshape(128, 1)
    f1 = jnp.pad(wf1.reshape(16, 5, 5, 120),
                 ((0, 0), (0, 0), (0, 3), (0, 0)))
    f1 = jnp.transpose(f1, (1, 0, 2, 3)).reshape(640, 120).T
    f2 = wf2.T
    f3 = wf3.T
    c1 = bf1.reshape(120, 1)
    c2 = bf2.reshape(84, 1)
    c3 = bf3.reshape(10, 1)

    def whole(shape):
        nd = len(shape)
        return pl.BlockSpec(shape, lambda i, nd=nd: (0,) * nd)

    out = pl.pallas_call(
        _fused_kernel,
        out_shape=jax.ShapeDtypeStruct((10, n_pad), jnp.float32),
        grid=(n_pad // _BN,),
        in_specs=[pl.BlockSpec((32, 3, 32, _BN), lambda i: (0, 0, 0, i)),
                  whole(t1.shape), whole(b1s.shape),
                  whole(t2.shape), whole(b2s.shape),
                  whole(f1.shape), whole(c1.shape),
                  whole(f2.shape), whole(c2.shape),
                  whole(f3.shape), whole(c3.shape)],
        out_specs=pl.BlockSpec((10, _BN), lambda i: (0, i)),
        compiler_params=pltpu.CompilerParams(
            dimension_semantics=("parallel",)),
    )(xt, t1, b1s, t2, b2s, f1, c1, f2, c2, f3, c3)
    return out.T[:n]


# speedup vs baseline: 1.3208x; 1.1502x over previous
import numpy as np

import jax
import jax.numpy as jnp
from jax.experimental import pallas as pl
from jax.experimental.pallas import tpu as pltpu

_BN = 128


def _tap_select(cin):
    sel = np.zeros((288, cin, 6, 6), np.float32)
    for a in range(6):
        for d in range(6):
            base = ((a // 2) * 3 + (d // 2)) * 4 + 2 * (a % 2) + d % 2
            for c in range(cin):
                sel[base * 8 + c, c, a, d] = 1.0
    return sel


def _col_select(win_w, kpad, k_max):
    w = np.arange(win_w)[:, None, None]
    d = np.arange(6)[None, :, None]
    k = np.arange(kpad)[None, None, :]
    return ((w == 2 * k + d) & (k <= k_max)).astype(np.float32)


def _toeplitz(w_packed, cin, win_w, kpad, k_max):
    cout = w_packed.shape[1]
    t = jnp.einsum('qot,tcad,wdk->qokacw', w_packed,
                   _tap_select(cin), _col_select(win_w, kpad, k_max))
    return t.reshape(4 * cout * kpad, 6 * cin * win_w)


def _fused_kernel(x_ref, t1_ref, b1_ref, t2_ref, b2_ref,
                  f1_ref, c1_ref, f2_ref, c2_ref, f3_ref, c3_ref, o_ref):
    bn = o_ref.shape[1]
    X = x_ref[...].reshape(32, 96, bn)

    t1 = t1_ref[...]
    rows = []
    for h in range(14):
        win = X[2 * h:2 * h + 6].reshape(576, bn)
        z = jnp.dot(t1, win, preferred_element_type=jnp.float32)
        rows.append(jnp.max(z.reshape(4, 96, bn), axis=0))
    y = jnp.stack(rows, axis=0)
    p1 = jnp.maximum(y + b1_ref[...], 0.0).astype(jnp.bfloat16)

    t2 = t2_ref[...]
    rows2 = []
    for h in range(5):
        win = p1[2 * h:2 * h + 6].reshape(576, bn)
        z = jnp.dot(t2, win, preferred_element_type=jnp.float32)
        rows2.append(jnp.max(z.reshape(4, 128, bn), axis=0))
    y2 = jnp.stack(rows2, axis=0)
    p2 = jnp.maximum(y2 + b2_ref[...], 0.0).astype(jnp.bfloat16)
    flat = p2.reshape(640, bn)

    h1 = jnp.dot(f1_ref[...], flat, preferred_element_type=jnp.float32)
    h1 = jnp.maximum(h1 + c1_ref[...], 0.0).astype(jnp.bfloat16)
    h2 = jnp.dot(f2_ref[...], h1, preferred_element_type=jnp.float32)
    h2 = jnp.maximum(h2 + c2_ref[...], 0.0).astype(jnp.bfloat16)
    z3 = jnp.dot(f3_ref[...], h2, preferred_element_type=jnp.float32)
    z3 = z3 + c3_ref[...]
    m = jnp.max(z3, axis=0, keepdims=True)
    e = jnp.exp(z3 - m)
    s = jnp.sum(e, axis=0, keepdims=True)
    o_ref[...] = z3 - m - jnp.log(s)


def kernel(x, w1, b1, w2, b2, wf1, bf1, wf2, bf2, wf3, bf3):
    n = x.shape[0]
    n_pad = ((n + _BN - 1) // _BN) * _BN
    if n_pad != n:
        x = jnp.pad(x, ((0, n_pad - n), (0, 0), (0, 0), (0, 0)))

    xt = jnp.transpose(x, (2, 1, 3, 0)).astype(jnp.bfloat16)

    t1 = _toeplitz(w1, 3, 32, 16, 13).astype(jnp.bfloat16)
    t2 = _toeplitz(w2, 6, 16, 8, 4).astype(jnp.bfloat16)
    b1s = (jnp.tile(b1, (1, 16)) *
           (np.arange(16) < 14).astype(np.float32)).reshape(96, 1)
    b2s = (jnp.tile(b2, (1, 8)) *
           (np.arange(8) < 5).astype(np.float32)).reshape(128, 1)
    f1 = jnp.pad(wf1.reshape(16, 5, 5, 120),
                 ((0, 0), (0, 0), (0, 3), (0, 0)))
    f1 = jnp.transpose(f1, (1, 0, 2, 3)).reshape(640, 120).T.astype(jnp.bfloat16)
    f2 = wf2.T.astype(jnp.bfloat16)
    f3 = wf3.T.astype(jnp.bfloat16)
    c1 = bf1.reshape(120, 1)
    c2 = bf2.reshape(84, 1)
    c3 = bf3.reshape(10, 1)

    def whole(shape):
        nd = len(shape)
        return pl.BlockSpec(shape, lambda i, nd=nd: (0,) * nd)

    out = pl.pallas_call(
        _fused_kernel,
        out_shape=jax.ShapeDtypeStruct((10, n_pad), jnp.float32),
        grid=(n_pad // _BN,),
        in_specs=[pl.BlockSpec((32, 3, 32, _BN), lambda i: (0, 0, 0, i)),
                  whole(t1.shape), whole(b1s.shape),
                  whole(t2.shape), whole(b2s.shape),
                  whole(f1.shape), whole(c1.shape),
                  whole(f2.shape), whole(c2.shape),
                  whole(f3.shape), whole(c3.shape)],
        out_specs=pl.BlockSpec((10, _BN), lambda i: (0, i)),
        compiler_params=pltpu.CompilerParams(
            dimension_semantics=("parallel",)),
    )(xt, t1, b1s, t2, b2s, f1, c1, f2, c2, f3, c3)
    return out.T[:n]


# speedup vs baseline: 1.7565x; 1.3299x over previous
import numpy as np

import jax
import jax.numpy as jnp
from jax.experimental import pallas as pl
from jax.experimental.pallas import tpu as pltpu

_BN = 256


def _tap_select(cin):
    sel = np.zeros((288, cin, 6, 6), np.float32)
    for a in range(6):
        for d in range(6):
            base = ((a // 2) * 3 + (d // 2)) * 4 + 2 * (a % 2) + d % 2
            for c in range(cin):
                sel[base * 8 + c, c, a, d] = 1.0
    return sel


def _col_select(win_w, kpad, k_max):
    w = np.arange(win_w)[:, None, None]
    d = np.arange(6)[None, :, None]
    k = np.arange(kpad)[None, None, :]
    return ((w == 2 * k + d) & (k <= k_max)).astype(np.float32)


def _toeplitz(w_packed, cin, win_w, kpad, k_max):
    cout = w_packed.shape[1]
    t = jnp.einsum('qot,tcad,wdk->qokacw', w_packed,
                   _tap_select(cin), _col_select(win_w, kpad, k_max))
    return t.reshape(4 * cout * kpad, 6 * cin * win_w)


def _fused_kernel(x_ref, t1_ref, b1_ref, t2_ref, b2_ref,
                  f1_ref, c1_ref, f2_ref, c2_ref, f3_ref, c3_ref, o_ref):
    bn = o_ref.shape[1]
    X = x_ref[...].reshape(32, 96, bn)

    t1 = t1_ref[...]
    rows = []
    for h in range(14):
        win = X[2 * h:2 * h + 6].reshape(576, bn)
        z = jnp.dot(t1, win, preferred_element_type=jnp.float32)
        rows.append(jnp.max(z.reshape(4, 96, bn), axis=0))
    y = jnp.stack(rows, axis=0)
    p1 = jnp.maximum(y + b1_ref[...], 0.0).astype(jnp.bfloat16)

    t2 = t2_ref[...]
    rows2 = []
    for h in range(5):
        win = p1[2 * h:2 * h + 6].reshape(576, bn)
        z = jnp.dot(t2, win, preferred_element_type=jnp.float32)
        rows2.append(jnp.max(z.reshape(4, 128, bn), axis=0))
    y2 = jnp.stack(rows2, axis=0)
    p2 = jnp.maximum(y2 + b2_ref[...], 0.0).astype(jnp.bfloat16)
    flat = p2.reshape(640, bn)

    h1 = jnp.dot(f1_ref[...], flat, preferred_element_type=jnp.float32)
    h1 = jnp.maximum(h1 + c1_ref[...], 0.0).astype(jnp.bfloat16)
    h2 = jnp.dot(f2_ref[...], h1, preferred_element_type=jnp.float32)
    h2 = jnp.maximum(h2 + c2_ref[...], 0.0).astype(jnp.bfloat16)
    z3 = jnp.dot(f3_ref[...], h2, preferred_element_type=jnp.float32)
    z3 = z3 + c3_ref[...]
    m = jnp.max(z3, axis=0, keepdims=True)
    e = jnp.exp(z3 - m)
    s = jnp.sum(e, axis=0, keepdims=True)
    o_ref[...] = z3 - m - jnp.log(s)


def kernel(x, w1, b1, w2, b2, wf1, bf1, wf2, bf2, wf3, bf3):
    n = x.shape[0]
    n_pad = ((n + _BN - 1) // _BN) * _BN
    if n_pad != n:
        x = jnp.pad(x, ((0, n_pad - n), (0, 0), (0, 0), (0, 0)))

    xt = jnp.transpose(x, (2, 1, 3, 0)).astype(jnp.bfloat16)

    t1 = _toeplitz(w1, 3, 32, 16, 13).astype(jnp.bfloat16)
    t2 = _toeplitz(w2, 6, 16, 8, 4).astype(jnp.bfloat16)
    b1s = (jnp.tile(b1, (1, 16)) *
           (np.arange(16) < 14).astype(np.float32)).reshape(96, 1)
    b2s = (jnp.tile(b2, (1, 8)) *
           (np.arange(8) < 5).astype(np.float32)).reshape(128, 1)
    f1 = jnp.pad(wf1.reshape(16, 5, 5, 120),
                 ((0, 0), (0, 0), (0, 3), (0, 0)))
    f1 = jnp.transpose(f1, (1, 0, 2, 3)).reshape(640, 120).T.astype(jnp.bfloat16)
    f2 = wf2.T.astype(jnp.bfloat16)
    f3 = wf3.T.astype(jnp.bfloat16)
    c1 = bf1.reshape(120, 1)
    c2 = bf2.reshape(84, 1)
    c3 = bf3.reshape(10, 1)

    def whole(shape):
        nd = len(shape)
        return pl.BlockSpec(shape, lambda i, nd=nd: (0,) * nd)

    out = pl.pallas_call(
        _fused_kernel,
        out_shape=jax.ShapeDtypeStruct((10, n_pad), jnp.float32),
        grid=(n_pad // _BN,),
        in_specs=[pl.BlockSpec((32, 3, 32, _BN), lambda i: (0, 0, 0, i)),
                  whole(t1.shape), whole(b1s.shape),
                  whole(t2.shape), whole(b2s.shape),
                  whole(f1.shape), whole(c1.shape),
                  whole(f2.shape), whole(c2.shape),
                  whole(f3.shape), whole(c3.shape)],
        out_specs=pl.BlockSpec((10, _BN), lambda i: (0, i)),
        compiler_params=pltpu.CompilerParams(
            dimension_semantics=("parallel",)),
    )(xt, t1, b1s, t2, b2s, f1, c1, f2, c2, f3, c3)
    return out.T[:n]


# speedup vs baseline: 1.8554x; 1.0563x over previous
import numpy as np

import jax
import jax.numpy as jnp
from jax.experimental import pallas as pl
from jax.experimental.pallas import tpu as pltpu

_BN = 512


def _tap_select(cin):
    sel = np.zeros((288, cin, 6, 6), np.float32)
    for a in range(6):
        for d in range(6):
            base = ((a // 2) * 3 + (d // 2)) * 4 + 2 * (a % 2) + d % 2
            for c in range(cin):
                sel[base * 8 + c, c, a, d] = 1.0
    return sel


def _col_select(win_w, kpad, k_max):
    w = np.arange(win_w)[:, None, None]
    d = np.arange(6)[None, :, None]
    k = np.arange(kpad)[None, None, :]
    return ((w == 2 * k + d) & (k <= k_max)).astype(np.float32)


def _toeplitz(w_packed, cin, win_w, kpad, k_max):
    cout = w_packed.shape[1]
    t = jnp.einsum('qot,tcad,wdk->qokacw', w_packed,
                   _tap_select(cin), _col_select(win_w, kpad, k_max))
    return t.reshape(4 * cout * kpad, 6 * cin * win_w)


def _fused_kernel(x_ref, t1_ref, b1_ref, t2_ref, b2_ref,
                  f1_ref, c1_ref, f2_ref, c2_ref, f3_ref, c3_ref, o_ref):
    bn = o_ref.shape[1]
    X = x_ref[...].reshape(32, 96, bn)

    t1 = t1_ref[...]
    rows = []
    for h in range(14):
        win = X[2 * h:2 * h + 6].reshape(576, bn)
        z = jnp.dot(t1, win, preferred_element_type=jnp.float32)
        rows.append(jnp.max(z.reshape(4, 96, bn), axis=0))
    y = jnp.stack(rows, axis=0)
    p1 = jnp.maximum(y + b1_ref[...], 0.0).astype(jnp.bfloat16)

    t2 = t2_ref[...]
    rows2 = []
    for h in range(5):
        win = p1[2 * h:2 * h + 6].reshape(576, bn)
        z = jnp.dot(t2, win, preferred_element_type=jnp.float32)
        rows2.append(jnp.max(z.reshape(4, 128, bn), axis=0))
    y2 = jnp.stack(rows2, axis=0)
    p2 = jnp.maximum(y2 + b2_ref[...], 0.0).astype(jnp.bfloat16)
    flat = p2.reshape(640, bn)

    h1 = jnp.dot(f1_ref[...], flat, preferred_element_type=jnp.float32)
    h1 = jnp.maximum(h1 + c1_ref[...], 0.0).astype(jnp.bfloat16)
    h2 = jnp.dot(f2_ref[...], h1, preferred_element_type=jnp.float32)
    h2 = jnp.maximum(h2 + c2_ref[...], 0.0).astype(jnp.bfloat16)
    z3 = jnp.dot(f3_ref[...], h2, preferred_element_type=jnp.float32)
    z3 = z3 + c3_ref[...]
    m = jnp.max(z3, axis=0, keepdims=True)
    e = jnp.exp(z3 - m)
    s = jnp.sum(e, axis=0, keepdims=True)
    o_ref[...] = z3 - m - jnp.log(s)


def kernel(x, w1, b1, w2, b2, wf1, bf1, wf2, bf2, wf3, bf3):
    n = x.shape[0]
    n_pad = ((n + _BN - 1) // _BN) * _BN
    if n_pad != n:
        x = jnp.pad(x, ((0, n_pad - n), (0, 0), (0, 0), (0, 0)))

    xt = jnp.transpose(x, (2, 1, 3, 0)).astype(jnp.bfloat16)

    t1 = _toeplitz(w1, 3, 32, 16, 13).astype(jnp.bfloat16)
    t2 = _toeplitz(w2, 6, 16, 8, 4).astype(jnp.bfloat16)
    b1s = (jnp.tile(b1, (1, 16)) *
           (np.arange(16) < 14).astype(np.float32)).reshape(96, 1)
    b2s = (jnp.tile(b2, (1, 8)) *
           (np.arange(8) < 5).astype(np.float32)).reshape(128, 1)
    f1 = jnp.pad(wf1.reshape(16, 5, 5, 120),
                 ((0, 0), (0, 0), (0, 3), (0, 0)))
    f1 = jnp.transpose(f1, (1, 0, 2, 3)).reshape(640, 120).T.astype(jnp.bfloat16)
    f2 = wf2.T.astype(jnp.bfloat16)
    f3 = wf3.T.astype(jnp.bfloat16)
    c1 = bf1.reshape(120, 1)
    c2 = bf2.reshape(84, 1)
    c3 = bf3.reshape(10, 1)

    def whole(shape):
        nd = len(shape)
        return pl.BlockSpec(shape, lambda i, nd=nd: (0,) * nd)

    out = pl.pallas_call(
        _fused_kernel,
        out_shape=jax.ShapeDtypeStruct((10, n_pad), jnp.float32),
        grid=(n_pad // _BN,),
        in_specs=[pl.BlockSpec((32, 3, 32, _BN), lambda i: (0, 0, 0, i)),
                  whole(t1.shape), whole(b1s.shape),
                  whole(t2.shape), whole(b2s.shape),
                  whole(f1.shape), whole(c1.shape),
                  whole(f2.shape), whole(c2.shape),
                  whole(f3.shape), whole(c3.shape)],
        out_specs=pl.BlockSpec((10, _BN), lambda i: (0, i)),
        compiler_params=pltpu.CompilerParams(
            dimension_semantics=("parallel",)),
    )(xt, t1, b1s, t2, b2s, f1, c1, f2, c2, f3, c3)
    return out.T[:n]


# speedup vs baseline: 1.8800x; 1.0133x over previous
import numpy as np

import jax
import jax.numpy as jnp
from jax.experimental import pallas as pl
from jax.experimental.pallas import tpu as pltpu

_BN = 1024


def _tap_select(cin):
    sel = np.zeros((288, cin, 6, 6), np.float32)
    for a in range(6):
        for d in range(6):
            base = ((a // 2) * 3 + (d // 2)) * 4 + 2 * (a % 2) + d % 2
            for c in range(cin):
                sel[base * 8 + c, c, a, d] = 1.0
    return sel


def _col_select(win_w, kpad, k_max):
    w = np.arange(win_w)[:, None, None]
    d = np.arange(6)[None, :, None]
    k = np.arange(kpad)[None, None, :]
    return ((w == 2 * k + d) & (k <= k_max)).astype(np.float32)


def _toeplitz(w_packed, cin, win_w, kpad, k_max):
    cout = w_packed.shape[1]
    t = jnp.einsum('qot,tcad,wdk->qokacw', w_packed,
                   _tap_select(cin), _col_select(win_w, kpad, k_max))
    return t.reshape(4 * cout * kpad, 6 * cin * win_w)


def _fused_kernel(x_ref, t1_ref, b1_ref, t2_ref, b2_ref,
                  f1_ref, c1_ref, f2_ref, c2_ref, f3_ref, c3_ref, o_ref):
    bn = o_ref.shape[1]
    X = x_ref[...].reshape(32, 96, bn)

    t1 = t1_ref[...]
    rows = []
    for h in range(14):
        win = X[2 * h:2 * h + 6].reshape(576, bn)
        z = jnp.dot(t1, win, preferred_element_type=jnp.float32)
        rows.append(jnp.max(z.reshape(4, 96, bn), axis=0))
    y = jnp.stack(rows, axis=0)
    p1 = jnp.maximum(y + b1_ref[...], 0.0).astype(jnp.bfloat16)

    t2 = t2_ref[...]
    rows2 = []
    for h in range(5):
        win = p1[2 * h:2 * h + 6].reshape(576, bn)
        z = jnp.dot(t2, win, preferred_element_type=jnp.float32)
        rows2.append(jnp.max(z.reshape(4, 128, bn), axis=0))
    y2 = jnp.stack(rows2, axis=0)
    p2 = jnp.maximum(y2 + b2_ref[...], 0.0).astype(jnp.bfloat16)
    flat = p2.reshape(640, bn)

    h1 = jnp.dot(f1_ref[...], flat, preferred_element_type=jnp.float32)
    h1 = jnp.maximum(h1 + c1_ref[...], 0.0).astype(jnp.bfloat16)
    h2 = jnp.dot(f2_ref[...], h1, preferred_element_type=jnp.float32)
    h2 = jnp.maximum(h2 + c2_ref[...], 0.0).astype(jnp.bfloat16)
    z3 = jnp.dot(f3_ref[...], h2, preferred_element_type=jnp.float32)
    z3 = z3 + c3_ref[...]
    m = jnp.max(z3, axis=0, keepdims=True)
    e = jnp.exp(z3 - m)
    s = jnp.sum(e, axis=0, keepdims=True)
    o_ref[...] = z3 - m - jnp.log(s)


def kernel(x, w1, b1, w2, b2, wf1, bf1, wf2, bf2, wf3, bf3):
    n = x.shape[0]
    n_pad = ((n + _BN - 1) // _BN) * _BN
    if n_pad != n:
        x = jnp.pad(x, ((0, n_pad - n), (0, 0), (0, 0), (0, 0)))

    xt = jnp.transpose(x, (2, 1, 3, 0)).astype(jnp.bfloat16)

    t1 = _toeplitz(w1, 3, 32, 16, 13).astype(jnp.bfloat16)
    t2 = _toeplitz(w2, 6, 16, 8, 4).astype(jnp.bfloat16)
    b1s = (jnp.tile(b1, (1, 16)) *
           (np.arange(16) < 14).astype(np.float32)).reshape(96, 1)
    b2s = (jnp.tile(b2, (1, 8)) *
           (np.arange(8) < 5).astype(np.float32)).reshape(128, 1)
    f1 = jnp.pad(wf1.reshape(16, 5, 5, 120),
                 ((0, 0), (0, 0), (0, 3), (0, 0)))
    f1 = jnp.transpose(f1, (1, 0, 2, 3)).reshape(640, 120).T.astype(jnp.bfloat16)
    f2 = wf2.T.astype(jnp.bfloat16)
    f3 = wf3.T.astype(jnp.bfloat16)
    c1 = bf1.reshape(120, 1)
    c2 = bf2.reshape(84, 1)
    c3 = bf3.reshape(10, 1)

    def whole(shape):
        nd = len(shape)
        return pl.BlockSpec(shape, lambda i, nd=nd: (0,) * nd)

    out = pl.pallas_call(
        _fused_kernel,
        out_shape=jax.ShapeDtypeStruct((10, n_pad), jnp.float32),
        grid=(n_pad // _BN,),
        in_specs=[pl.BlockSpec((32, 3, 32, _BN), lambda i: (0, 0, 0, i)),
                  whole(t1.shape), whole(b1s.shape),
                  whole(t2.shape), whole(b2s.shape),
                  whole(f1.shape), whole(c1.shape),
                  whole(f2.shape), whole(c2.shape),
                  whole(f3.shape), whole(c3.shape)],
        out_specs=pl.BlockSpec((10, _BN), lambda i: (0, i)),
        compiler_params=pltpu.CompilerParams(
            dimension_semantics=("parallel",)),
    )(xt, t1, b1s, t2, b2s, f1, c1, f2, c2, f3, c3)
    return out.T[:n]
